# Initial kernel scaffold; baseline (speedup 1.0000x reference)
import jax
import jax.numpy as jnp
from jax.experimental import pallas as pl


def kernel(pred_logits, pred_boxes, target_sizes):
    raise NotImplementedError("write your pallas kernel here")


# speedup vs baseline: 5.8363x; 5.8363x over previous
import jax
import jax.numpy as jnp
from jax.experimental import pallas as pl
from jax.experimental.pallas import tpu as pltpu

_B, _K, _C = 16, 20000, 91
_ROWS, _LCOLS = 160, 125
_M = 128
_TOPK = 100
_NEG = -1e30
_BIG = 2**30


def _body(logits_ref, boxes_ref, tsz_ref,
          scores_ref, labels_ref, boxes_out_ref,
          xm_ref, cand_ref, sel_ref, vsc_ref, vfi_ref, gb_ref,
          ids_smem, bid_smem):
    x = logits_ref[0]
    xm = x.reshape(_ROWS, _LCOLS, _C).max(axis=2)
    xm_ref[...] = jnp.concatenate(
        [xm, jnp.full((_ROWS, 128 - _LCOLS), _NEG, jnp.float32)], axis=1)

    row_i = jax.lax.broadcasted_iota(jnp.int32, (_ROWS, 128), 0)
    lane_i = jax.lax.broadcasted_iota(jnp.int32, (_ROWS, 128), 1)
    box_id_map = jnp.where(lane_i < _LCOLS, row_i * _LCOLS + lane_i, _BIG)

    def sel_step(i, carry):
        cur = xm_ref[...]
        m = jnp.max(cur, keepdims=True)
        bid = jnp.min(jnp.where(cur == m, box_id_map, _BIG),
                      keepdims=True)
        sel_ref[pl.ds(i, 1), :] = bid
        ids_smem[i] = bid[0, 0]
        xm_ref[...] = jnp.where(box_id_map == bid, _NEG, cur)
        return carry
    jax.lax.fori_loop(0, _M, sel_step, 0)

    cand_ref[...] = jnp.full((_M, 128), _NEG, jnp.float32)

    def gat_step(j, carry):
        bid = ids_smem[j]
        cand_ref[pl.ds(j, 1), 0:_C] = logits_ref[0, pl.ds(bid, 1), :]
        return carry
    jax.lax.fori_loop(0, _M, gat_step, 0)

    ids_v = sel_ref[...]
    clane = jax.lax.broadcasted_iota(jnp.int32, (_M, 128), 1)
    fidx_map = jnp.where(clane < _C, ids_v * _C + clane, _BIG)

    def top_step(i, carry):
        cur = cand_ref[...]
        m = jnp.max(cur, keepdims=True)
        fi = jnp.min(jnp.where(cur == m, fidx_map, _BIG),
                     keepdims=True)
        bid_smem[i] = fi[0, 0] // _C
        vsc_ref[pl.ds(i, 1), :] = m
        vfi_ref[pl.ds(i, 1), :] = fi
        cand_ref[...] = jnp.where(fidx_map == fi, _NEG, cur)
        return carry
    jax.lax.fori_loop(0, _TOPK, top_step, 0)

    scores_ref[0] = jax.nn.sigmoid(vsc_ref[...])
    labels_ref[0] = vfi_ref[...] % _C

    def box_step(i, carry):
        bid = bid_smem[i]
        gb_ref[pl.ds(i, 1), :] = boxes_ref[0, pl.ds(bid, 1), :]
        return carry
    jax.lax.fori_loop(0, _TOPK, box_step, 0)

    g = gb_ref[...]
    cx, cy = g[:, 0:1], g[:, 1:2]
    w, h = g[:, 2:3], g[:, 3:4]
    xyxy = jnp.concatenate(
        [cx - 0.5 * w, cy - 0.5 * h, cx + 0.5 * w, cy + 0.5 * h], axis=1)
    t = tsz_ref[0].astype(jnp.float32)
    scale = jnp.concatenate([t[:, 1:2], t[:, 0:1], t[:, 1:2], t[:, 0:1]],
                            axis=1)
    boxes_out_ref[0] = xyxy * scale


def kernel(pred_logits, pred_boxes, target_sizes):
    B = pred_logits.shape[0]
    tsz3 = target_sizes.reshape(B, 1, 2)
    scores3, labels3, boxes = pl.pallas_call(
        _body,
        grid=(B,),
        in_specs=[
            pl.BlockSpec((1, _K, _C), lambda b: (b, 0, 0)),
            pl.BlockSpec((1, _K, 4), lambda b: (b, 0, 0)),
            pl.BlockSpec((1, 1, 2), lambda b: (b, 0, 0)),
        ],
        out_specs=[
            pl.BlockSpec((1, _TOPK, 1), lambda b: (b, 0, 0)),
            pl.BlockSpec((1, _TOPK, 1), lambda b: (b, 0, 0)),
            pl.BlockSpec((1, _TOPK, 4), lambda b: (b, 0, 0)),
        ],
        out_shape=[
            jax.ShapeDtypeStruct((B, _TOPK, 1), jnp.float32),
            jax.ShapeDtypeStruct((B, _TOPK, 1), jnp.int32),
            jax.ShapeDtypeStruct((B, _TOPK, 4), jnp.float32),
        ],
        scratch_shapes=[
            pltpu.VMEM((_ROWS, 128), jnp.float32),
            pltpu.VMEM((_M, 128), jnp.float32),
            pltpu.VMEM((_M, 1), jnp.int32),
            pltpu.VMEM((_TOPK, 1), jnp.float32),
            pltpu.VMEM((_TOPK, 1), jnp.int32),
            pltpu.VMEM((_TOPK, 4), jnp.float32),
            pltpu.SMEM((_M,), jnp.int32),
            pltpu.SMEM((_TOPK,), jnp.int32),
        ],
        compiler_params=pltpu.CompilerParams(
            dimension_semantics=("arbitrary",)),
        interpret=False,
    )(pred_logits, pred_boxes, tsz3)
    return (scores3.reshape(B, _TOPK), labels3.reshape(B, _TOPK), boxes)
